# baseline (speedup 1.0000x reference)
import functools

import jax
import jax.numpy as jnp
from jax import lax
from jax.experimental import pallas as pl
from jax.experimental.pallas import tpu as pltpu
from jax.experimental.pallas import tpu_sc as plsc

K = 8192
D = 32
BETA = 0.25

TBLK = 512
KBLK = 1024
NKB = K // KBLK

N_TOKENS = 16 * 32 * 32

NC = 2
NS = 16
NW = NC * NS
BPW = N_TOKENS // NW


NKB_HALF = NKB // 2


def _argmin_body(x_ref, xx_ref, w_ref, ww_ref, out_ref,
                 best0_ref, besti0_ref, best1_ref, besti1_ref):
    j = pl.program_id(1)

    @pl.when(j == 0)
    def _init():
        best0_ref[...] = jnp.full((TBLK, 1), jnp.inf, jnp.float32)
        besti0_ref[...] = jnp.zeros((TBLK, 1), jnp.int32)
        best1_ref[...] = jnp.full((TBLK, 1), jnp.inf, jnp.float32)
        besti1_ref[...] = jnp.zeros((TBLK, 1), jnp.int32)

    x = x_ref[...]
    w = w_ref[...]
    xw = lax.dot_general(x, w, (((1,), (1,)), ((), ())),
                         preferred_element_type=jnp.float32)
    dist = (xx_ref[...] - 2.0 * xw) + ww_ref[...]
    m = jnp.min(dist, axis=1, keepdims=True)
    iota = lax.broadcasted_iota(jnp.int32, (TBLK, KBLK), 1)
    im = jnp.min(jnp.where(dist == m, iota, jnp.int32(2**30)),
                 axis=1, keepdims=True) + j * KBLK

    @pl.when(j < NKB_HALF)
    def _upd0():
        cond = m < best0_ref[...]
        best0_ref[...] = jnp.where(cond, m, best0_ref[...])
        besti0_ref[...] = jnp.where(cond, im, besti0_ref[...])

    @pl.when(j >= NKB_HALF)
    def _upd1():
        cond = m < best1_ref[...]
        best1_ref[...] = jnp.where(cond, m, best1_ref[...])
        besti1_ref[...] = jnp.where(cond, im, besti1_ref[...])

    @pl.when(j == NKB - 1)
    def _flush():
        u = lax.bitcast_convert_type(best0_ref[...], jnp.int32)
        b0 = lax.bitcast_convert_type((u + 0x7FFF) >> 16 << 16, jnp.float32)
        take0 = b0 <= best1_ref[...]
        out_ref[...] = jnp.where(take0, besti0_ref[...], besti1_ref[...])


def _tc_argmin(flat, xx, W, ww_row):
    grid = (N_TOKENS // TBLK, NKB)
    return pl.pallas_call(
        _argmin_body,
        grid=grid,
        in_specs=[
            pl.BlockSpec((TBLK, D), lambda i, j: (i, 0)),
            pl.BlockSpec((TBLK, 1), lambda i, j: (i, 0)),
            pl.BlockSpec((KBLK, D), lambda i, j: (j, 0)),
            pl.BlockSpec((1, KBLK), lambda i, j: (0, j)),
        ],
        out_specs=pl.BlockSpec((TBLK, 1), lambda i, j: (i, 0)),
        out_shape=jax.ShapeDtypeStruct((N_TOKENS, 1), jnp.int32),
        scratch_shapes=[
            pltpu.VMEM((TBLK, 1), jnp.float32),
            pltpu.VMEM((TBLK, 1), jnp.int32),
            pltpu.VMEM((TBLK, 1), jnp.float32),
            pltpu.VMEM((TBLK, 1), jnp.int32),
        ],
        compiler_params=pltpu.CompilerParams(
            dimension_semantics=("parallel", "arbitrary"),
        ),
    )(flat, xx, W, ww_row)


def _sc_gather_body(w_hbm, idx_hbm, x_hbm, out_hbm, idx_v, rows_v, x_v, sem):
    wid = lax.axis_index("s") * NC + lax.axis_index("c")
    base = wid * BPW
    pltpu.sync_copy(idx_hbm.at[pl.ds(base, BPW)], idx_v)
    pltpu.async_copy(w_hbm.at[idx_v], rows_v, sem).wait()
    pltpu.sync_copy(x_hbm.at[pl.ds(base, BPW)], x_v)

    def body(i, carry):
        for j in (0, 16):
            xs = x_v[i, pl.ds(j, 16)]
            qs = rows_v[i, pl.ds(j, 16)]
            rows_v[i, pl.ds(j, 16)] = xs + BETA * (qs - xs)
        return carry

    lax.fori_loop(0, BPW, body, 0)
    pltpu.sync_copy(rows_v, out_hbm.at[pl.ds(base, BPW)])


@functools.cache
def _sc_gather_lerp():
    return pl.kernel(
        _sc_gather_body,
        out_type=jax.ShapeDtypeStruct((N_TOKENS, D), jnp.float32),
        mesh=plsc.VectorSubcoreMesh(core_axis_name="c", subcore_axis_name="s"),
        scratch_types=[
            pltpu.VMEM((BPW,), jnp.int32),
            pltpu.VMEM((BPW, D), jnp.float32),
            pltpu.VMEM((BPW, D), jnp.float32),
            pltpu.SemaphoreType.DMA,
        ],
        compiler_params=pltpu.CompilerParams(use_tc_tiling_on_sc=False),
    )


def kernel(latents, W):
    x4 = jnp.transpose(latents, (0, 2, 3, 1))
    flat = x4.reshape(-1, D)
    xx = jnp.sum(flat ** 2, axis=1, keepdims=True)
    ww = jnp.sum(W ** 2, axis=1)
    idx = _tc_argmin(flat, xx, W, ww.reshape(1, K))
    idx_flat = idx.reshape(-1)
    q = _sc_gather_lerp()(W, idx_flat, flat)
    quantized = jnp.transpose(q.reshape(x4.shape), (0, 3, 1, 2))
    embed_ind = idx_flat.reshape(x4.shape[:-1])
    return (quantized, embed_ind)


# speedup vs baseline: 1.3208x; 1.3208x over previous
import functools

import jax
import jax.numpy as jnp
from jax import lax
from jax.experimental import pallas as pl
from jax.experimental.pallas import tpu as pltpu
from jax.experimental.pallas import tpu_sc as plsc

K = 8192
D = 32
BETA = 0.25

TBLK = 1024
KBLK = 2048
NKB = K // KBLK

N_TOKENS = 16 * 32 * 32

NC = 2
NS = 16
NW = NC * NS
BPW = N_TOKENS // NW


NKB_HALF = NKB // 2


def _argmin_body(x_ref, xx_ref, w_ref, ww_ref, out_ref,
                 best0_ref, besti0_ref, best1_ref, besti1_ref):
    j = pl.program_id(1)

    @pl.when(j == 0)
    def _init():
        best0_ref[...] = jnp.full((TBLK, 1), jnp.inf, jnp.float32)
        besti0_ref[...] = jnp.zeros((TBLK, 1), jnp.int32)
        best1_ref[...] = jnp.full((TBLK, 1), jnp.inf, jnp.float32)
        besti1_ref[...] = jnp.zeros((TBLK, 1), jnp.int32)

    x = x_ref[...]
    w = w_ref[...]
    xw = lax.dot_general(x, w, (((1,), (1,)), ((), ())),
                         preferred_element_type=jnp.float32)
    dist = (xx_ref[...] - 2.0 * xw) + ww_ref[...]
    m = jnp.min(dist, axis=1, keepdims=True)
    iota = lax.broadcasted_iota(jnp.int32, (TBLK, KBLK), 1)
    im = jnp.min(jnp.where(dist == m, iota, jnp.int32(2**30)),
                 axis=1, keepdims=True) + j * KBLK

    @pl.when(j < NKB_HALF)
    def _upd0():
        cond = m < best0_ref[...]
        best0_ref[...] = jnp.where(cond, m, best0_ref[...])
        besti0_ref[...] = jnp.where(cond, im, besti0_ref[...])

    @pl.when(j >= NKB_HALF)
    def _upd1():
        cond = m < best1_ref[...]
        best1_ref[...] = jnp.where(cond, m, best1_ref[...])
        besti1_ref[...] = jnp.where(cond, im, besti1_ref[...])

    @pl.when(j == NKB - 1)
    def _flush():
        u = lax.bitcast_convert_type(best0_ref[...], jnp.int32)
        b0 = lax.bitcast_convert_type((u + 0x7FFF) >> 16 << 16, jnp.float32)
        take0 = b0 <= best1_ref[...]
        out_ref[...] = jnp.where(take0, besti0_ref[...], besti1_ref[...])


def _tc_argmin(flat, xx, W, ww_row):
    grid = (N_TOKENS // TBLK, NKB)
    return pl.pallas_call(
        _argmin_body,
        grid=grid,
        in_specs=[
            pl.BlockSpec((TBLK, D), lambda i, j: (i, 0)),
            pl.BlockSpec((TBLK, 1), lambda i, j: (i, 0)),
            pl.BlockSpec((KBLK, D), lambda i, j: (j, 0)),
            pl.BlockSpec((1, KBLK), lambda i, j: (0, j)),
        ],
        out_specs=pl.BlockSpec((TBLK, 1), lambda i, j: (i, 0)),
        out_shape=jax.ShapeDtypeStruct((N_TOKENS, 1), jnp.int32),
        scratch_shapes=[
            pltpu.VMEM((TBLK, 1), jnp.float32),
            pltpu.VMEM((TBLK, 1), jnp.int32),
            pltpu.VMEM((TBLK, 1), jnp.float32),
            pltpu.VMEM((TBLK, 1), jnp.int32),
        ],
        compiler_params=pltpu.CompilerParams(
            dimension_semantics=("parallel", "arbitrary"),
        ),
    )(flat, xx, W, ww_row)


def _sc_gather_body(w_hbm, idx_hbm, x_hbm, out_hbm, idx_v, rows_v, x_v, sem):
    wid = lax.axis_index("s") * NC + lax.axis_index("c")
    base = wid * BPW
    pltpu.sync_copy(idx_hbm.at[pl.ds(base, BPW)], idx_v)
    pltpu.async_copy(w_hbm.at[idx_v], rows_v, sem).wait()
    pltpu.sync_copy(x_hbm.at[pl.ds(base, BPW)], x_v)

    def body(i, carry):
        for j in (0, 16):
            xs = x_v[i, pl.ds(j, 16)]
            qs = rows_v[i, pl.ds(j, 16)]
            rows_v[i, pl.ds(j, 16)] = xs + BETA * (qs - xs)
        return carry

    lax.fori_loop(0, BPW, body, 0)
    pltpu.sync_copy(rows_v, out_hbm.at[pl.ds(base, BPW)])


@functools.cache
def _sc_gather_lerp():
    return pl.kernel(
        _sc_gather_body,
        out_type=jax.ShapeDtypeStruct((N_TOKENS, D), jnp.float32),
        mesh=plsc.VectorSubcoreMesh(core_axis_name="c", subcore_axis_name="s"),
        scratch_types=[
            pltpu.VMEM((BPW,), jnp.int32),
            pltpu.VMEM((BPW, D), jnp.float32),
            pltpu.VMEM((BPW, D), jnp.float32),
            pltpu.SemaphoreType.DMA,
        ],
        compiler_params=pltpu.CompilerParams(use_tc_tiling_on_sc=False),
    )


def kernel(latents, W):
    x4 = jnp.transpose(latents, (0, 2, 3, 1))
    flat = x4.reshape(-1, D)
    xx = jnp.sum(flat ** 2, axis=1, keepdims=True)
    ww = jnp.sum(W ** 2, axis=1)
    idx = _tc_argmin(flat, xx, W, ww.reshape(1, K))
    idx_flat = idx.reshape(-1)
    q = _sc_gather_lerp()(W, idx_flat, flat)
    quantized = jnp.transpose(q.reshape(x4.shape), (0, 3, 1, 2))
    embed_ind = idx_flat.reshape(x4.shape[:-1])
    return (quantized, embed_ind)


# speedup vs baseline: 1.3641x; 1.0328x over previous
import functools

import jax
import jax.numpy as jnp
from jax import lax
from jax.experimental import pallas as pl
from jax.experimental.pallas import tpu as pltpu
from jax.experimental.pallas import tpu_sc as plsc

K = 8192
D = 32
BETA = 0.25

TBLK = 2048
KBLK = 2048
NKB = K // KBLK

N_TOKENS = 16 * 32 * 32

NC = 2
NS = 16
NW = NC * NS
BPW = N_TOKENS // NW


NKB_HALF = NKB // 2


def _argmin_body(x_ref, xx_ref, w_ref, ww_ref, out_ref,
                 best0_ref, besti0_ref, best1_ref, besti1_ref):
    j = pl.program_id(1)

    @pl.when(j == 0)
    def _init():
        best0_ref[...] = jnp.full((TBLK, 1), jnp.inf, jnp.float32)
        besti0_ref[...] = jnp.zeros((TBLK, 1), jnp.int32)
        best1_ref[...] = jnp.full((TBLK, 1), jnp.inf, jnp.float32)
        besti1_ref[...] = jnp.zeros((TBLK, 1), jnp.int32)

    x = x_ref[...]
    w = w_ref[...]
    xw = lax.dot_general(x, w, (((1,), (1,)), ((), ())),
                         preferred_element_type=jnp.float32)
    dist = (xx_ref[...] - 2.0 * xw) + ww_ref[...]
    m = jnp.min(dist, axis=1, keepdims=True)
    iota = lax.broadcasted_iota(jnp.int32, (TBLK, KBLK), 1)
    im = jnp.min(jnp.where(dist == m, iota, jnp.int32(2**30)),
                 axis=1, keepdims=True) + j * KBLK

    @pl.when(j < NKB_HALF)
    def _upd0():
        cond = m < best0_ref[...]
        best0_ref[...] = jnp.where(cond, m, best0_ref[...])
        besti0_ref[...] = jnp.where(cond, im, besti0_ref[...])

    @pl.when(j >= NKB_HALF)
    def _upd1():
        cond = m < best1_ref[...]
        best1_ref[...] = jnp.where(cond, m, best1_ref[...])
        besti1_ref[...] = jnp.where(cond, im, besti1_ref[...])

    @pl.when(j == NKB - 1)
    def _flush():
        u = lax.bitcast_convert_type(best0_ref[...], jnp.int32)
        b0 = lax.bitcast_convert_type((u + 0x7FFF) >> 16 << 16, jnp.float32)
        take0 = b0 <= best1_ref[...]
        out_ref[...] = jnp.where(take0, besti0_ref[...], besti1_ref[...])


def _tc_argmin(flat, xx, W, ww_row):
    grid = (N_TOKENS // TBLK, NKB)
    return pl.pallas_call(
        _argmin_body,
        grid=grid,
        in_specs=[
            pl.BlockSpec((TBLK, D), lambda i, j: (i, 0)),
            pl.BlockSpec((TBLK, 1), lambda i, j: (i, 0)),
            pl.BlockSpec((KBLK, D), lambda i, j: (j, 0)),
            pl.BlockSpec((1, KBLK), lambda i, j: (0, j)),
        ],
        out_specs=pl.BlockSpec((TBLK, 1), lambda i, j: (i, 0)),
        out_shape=jax.ShapeDtypeStruct((N_TOKENS, 1), jnp.int32),
        scratch_shapes=[
            pltpu.VMEM((TBLK, 1), jnp.float32),
            pltpu.VMEM((TBLK, 1), jnp.int32),
            pltpu.VMEM((TBLK, 1), jnp.float32),
            pltpu.VMEM((TBLK, 1), jnp.int32),
        ],
        compiler_params=pltpu.CompilerParams(
            dimension_semantics=("parallel", "arbitrary"),
        ),
    )(flat, xx, W, ww_row)


def _sc_gather_body(w_hbm, idx_hbm, x_hbm, out_hbm, idx_v, rows_v, x_v, sem):
    wid = lax.axis_index("s") * NC + lax.axis_index("c")
    base = wid * BPW
    pltpu.sync_copy(idx_hbm.at[pl.ds(base, BPW)], idx_v)
    pltpu.async_copy(w_hbm.at[idx_v], rows_v, sem).wait()
    pltpu.sync_copy(x_hbm.at[pl.ds(base, BPW)], x_v)

    def body(i, carry):
        for j in (0, 16):
            xs = x_v[i, pl.ds(j, 16)]
            qs = rows_v[i, pl.ds(j, 16)]
            rows_v[i, pl.ds(j, 16)] = xs + BETA * (qs - xs)
        return carry

    lax.fori_loop(0, BPW, body, 0)
    pltpu.sync_copy(rows_v, out_hbm.at[pl.ds(base, BPW)])


@functools.cache
def _sc_gather_lerp():
    return pl.kernel(
        _sc_gather_body,
        out_type=jax.ShapeDtypeStruct((N_TOKENS, D), jnp.float32),
        mesh=plsc.VectorSubcoreMesh(core_axis_name="c", subcore_axis_name="s"),
        scratch_types=[
            pltpu.VMEM((BPW,), jnp.int32),
            pltpu.VMEM((BPW, D), jnp.float32),
            pltpu.VMEM((BPW, D), jnp.float32),
            pltpu.SemaphoreType.DMA,
        ],
        compiler_params=pltpu.CompilerParams(use_tc_tiling_on_sc=False),
    )


def kernel(latents, W):
    x4 = jnp.transpose(latents, (0, 2, 3, 1))
    flat = x4.reshape(-1, D)
    xx = jnp.sum(flat ** 2, axis=1, keepdims=True)
    ww = jnp.sum(W ** 2, axis=1)
    idx = _tc_argmin(flat, xx, W, ww.reshape(1, K))
    idx_flat = idx.reshape(-1)
    q = _sc_gather_lerp()(W, idx_flat, flat)
    quantized = jnp.transpose(q.reshape(x4.shape), (0, 3, 1, 2))
    embed_ind = idx_flat.reshape(x4.shape[:-1])
    return (quantized, embed_ind)
